# Initial kernel scaffold; baseline (speedup 1.0000x reference)
import jax
import jax.numpy as jnp
from jax.experimental import pallas as pl


def kernel(pos_u, pos_v, neg_v, V):
    raise NotImplementedError("write your pallas kernel here")


# speedup vs baseline: 1.0047x; 1.0047x over previous
import functools

import jax
import jax.numpy as jnp
from jax import lax
from jax.experimental import pallas as pl
from jax.experimental.pallas import tpu as pltpu
from jax.experimental.pallas import tpu_sc as plsc

LANES = 16
NW = 32
NEG_MINOR = 80


def _sc_scores(V, u_w, pos_w, neg_w, n_pairs, bpw):
    D = V.shape[1]
    K = n_pairs - 1
    G = bpw // LANES
    neg_per_group = K * LANES
    nchunks = neg_per_group // NEG_MINOR
    rows_per_group = n_pairs * LANES

    mesh = plsc.VectorSubcoreMesh(core_axis_name="c", subcore_axis_name="s")

    @functools.partial(
        pl.kernel,
        mesh=mesh,
        compiler_params=pltpu.CompilerParams(
            needs_layout_passes=False, use_tc_tiling_on_sc=False
        ),
        out_type=jax.ShapeDtypeStruct((NW, n_pairs * bpw), jnp.float32),
        scratch_types=[
            pltpu.VMEM((G, LANES), jnp.int32),
            pltpu.VMEM((G * nchunks, NEG_MINOR), jnp.int32),
            pltpu.VMEM((bpw * D,), jnp.float32),
            pltpu.VMEM((rows_per_group, D), jnp.float32),
            pltpu.VMEM((rows_per_group, D), jnp.float32),
            pltpu.VMEM((n_pairs * bpw,), jnp.float32),
            pltpu.SemaphoreType.DMA,
            pltpu.SemaphoreType.DMA,
        ],
    )
    def k(V_hbm, u_hbm, pos_hbm, neg_hbm, out_hbm,
          pidx_v, nidx_v, u_v, rows0, rows1, sc_v, sem0, sem1):
        wid = lax.axis_index("s") * 2 + lax.axis_index("c")
        pltpu.sync_copy(pos_hbm.at[wid], pidx_v)
        pltpu.sync_copy(neg_hbm.at[wid], nidx_v)
        pltpu.sync_copy(u_hbm.at[wid], u_v)
        iota = lax.iota(jnp.int32, LANES)
        iota_k = iota * K
        iota_d = iota * D

        def issue(g, rows_v, sem):
            for c in range(nchunks):
                pltpu.async_copy(
                    V_hbm.at[nidx_v.at[g * nchunks + c]],
                    rows_v.at[pl.ds(c * NEG_MINOR, NEG_MINOR)],
                    sem,
                )
            pltpu.async_copy(
                V_hbm.at[pidx_v.at[g]],
                rows_v.at[pl.ds(neg_per_group, LANES)],
                sem,
            )

        def drain(rows_v, sem):
            for c in range(nchunks):
                pltpu.make_async_copy(
                    V_hbm.at[pl.ds(0, NEG_MINOR)],
                    rows_v.at[pl.ds(c * NEG_MINOR, NEG_MINOR)],
                    sem,
                ).wait()
            pltpu.make_async_copy(
                V_hbm.at[pl.ds(0, LANES)],
                rows_v.at[pl.ds(neg_per_group, LANES)],
                sem,
            ).wait()

        def compute(g, rows_v):
            def d_body(d, accs):
                dcol = (iota + d) & (D - 1)
                ud = plsc.load_gather(u_v, [g * (LANES * D) + iota_d + dcol])
                pos_r = plsc.load_gather(rows_v, [iota + neg_per_group, dcol])
                neg_rs = [
                    plsc.load_gather(rows_v, [iota_k + (j - 1), dcol])
                    for j in range(1, n_pairs)
                ]
                return (accs[0] + ud * pos_r,) + tuple(
                    accs[j] + ud * neg_rs[j - 1] for j in range(1, n_pairs)
                )

            zero = jnp.zeros((LANES,), jnp.float32)
            accs = lax.fori_loop(0, D, d_body, tuple(zero for _ in range(n_pairs)))
            sc_v[pl.ds(g * LANES, LANES)] = accs[0]
            for j in range(1, n_pairs):
                sc_v[pl.ds(j * bpw + g * LANES, LANES)] = -accs[j]

        issue(0, rows0, sem0)

        def pair_body(t, carry):
            g0 = 2 * t
            issue(g0 + 1, rows1, sem1)
            drain(rows0, sem0)
            compute(g0, rows0)

            @pl.when(g0 + 2 < G)
            def _():
                issue(g0 + 2, rows0, sem0)

            drain(rows1, sem1)
            compute(g0 + 1, rows1)
            return carry

        lax.fori_loop(0, G // 2, pair_body, 0)
        pltpu.sync_copy(sc_v, out_hbm.at[wid])

    return k(V, u_w, pos_w, neg_w)


def _tc_logsig_sum(scores2d):
    def body(x_ref, o_ref):
        x = x_ref[...]
        ls = jnp.minimum(x, 0.0) - jnp.log(1.0 + jnp.exp(-jnp.abs(x)))
        o_ref[...] = (-jnp.sum(ls)).reshape(1, 1)

    out = pl.pallas_call(
        body,
        out_shape=jax.ShapeDtypeStruct((1, 1), jnp.float32),
    )(scores2d)
    return out[0, 0]


def kernel(pos_u, pos_v, neg_v, V):
    B, D = pos_u.shape
    K = neg_v.shape[1]
    n_pairs = K + 1
    bpw = B // NW
    G = bpw // LANES

    pos_w = pos_v.astype(jnp.int32).reshape(NW, G, LANES)
    neg_w = neg_v.astype(jnp.int32).reshape(NW, -1, NEG_MINOR)
    u_w = pos_u.reshape(NW, bpw * D)
    scores = _sc_scores(V, u_w, pos_w, neg_w, n_pairs, bpw)
    return _tc_logsig_sum(scores.reshape(-1, 512))


# speedup vs baseline: 1.0768x; 1.0718x over previous
import functools

import jax
import jax.numpy as jnp
from jax import lax
from jax.experimental import pallas as pl
from jax.experimental.pallas import tpu as pltpu
from jax.experimental.pallas import tpu_sc as plsc

LANES = 16
NW = 32
NEG_MINOR = 80


def _sc_scores(V, u_w, pos_w, neg_w, n_pairs, bpw):
    DP = V.shape[1]
    D = u_w.shape[1] // bpw
    K = n_pairs - 1
    G = bpw // LANES
    neg_per_group = K * LANES
    nchunks = neg_per_group // NEG_MINOR
    rows_per_group = n_pairs * LANES

    mesh = plsc.VectorSubcoreMesh(core_axis_name="c", subcore_axis_name="s")

    @functools.partial(
        pl.kernel,
        mesh=mesh,
        compiler_params=pltpu.CompilerParams(
            needs_layout_passes=False, use_tc_tiling_on_sc=False
        ),
        out_type=jax.ShapeDtypeStruct((NW, n_pairs * bpw), jnp.float32),
        scratch_types=[
            pltpu.VMEM((G, LANES), jnp.int32),
            pltpu.VMEM((G * nchunks, NEG_MINOR), jnp.int32),
            pltpu.VMEM((LANES * D,), jnp.float32),
            pltpu.VMEM((LANES * D,), jnp.float32),
            pltpu.VMEM((rows_per_group, DP), jnp.float32),
            pltpu.VMEM((rows_per_group, DP), jnp.float32),
            pltpu.VMEM((n_pairs * bpw,), jnp.float32),
            pltpu.SemaphoreType.DMA,
            pltpu.SemaphoreType.DMA,
        ],
    )
    def k(V_hbm, u_hbm, pos_hbm, neg_hbm, out_hbm,
          pidx_v, nidx_v, u0, u1, rows0, rows1, sc_v, sem0, sem1):
        wid = lax.axis_index("s") * 2 + lax.axis_index("c")
        pltpu.sync_copy(pos_hbm.at[wid], pidx_v)
        pltpu.sync_copy(neg_hbm.at[wid], nidx_v)
        iota = lax.iota(jnp.int32, LANES)
        iota_k = iota * K
        iota_d = iota * D

        def issue(g, rows_v, u_v, sem):
            for c in range(nchunks):
                pltpu.async_copy(
                    V_hbm.at[nidx_v.at[g * nchunks + c]],
                    rows_v.at[pl.ds(c * NEG_MINOR, NEG_MINOR)],
                    sem,
                )
            pltpu.async_copy(
                V_hbm.at[pidx_v.at[g]],
                rows_v.at[pl.ds(neg_per_group, LANES)],
                sem,
            )
            pltpu.async_copy(
                u_hbm.at[wid, pl.ds(g * (LANES * D), LANES * D)],
                u_v,
                sem,
            )

        def drain(rows_v, u_v, sem):
            for c in range(nchunks):
                pltpu.make_async_copy(
                    V_hbm.at[pl.ds(0, NEG_MINOR)],
                    rows_v.at[pl.ds(c * NEG_MINOR, NEG_MINOR)],
                    sem,
                ).wait()
            pltpu.make_async_copy(
                V_hbm.at[pl.ds(0, LANES)],
                rows_v.at[pl.ds(neg_per_group, LANES)],
                sem,
            ).wait()
            pltpu.make_async_copy(
                u_hbm.at[wid, pl.ds(0, LANES * D)],
                u_v,
                sem,
            ).wait()

        def compute(g, rows_v, u_v):
            def d_body(d, accs):
                dcol = (iota + d) & (D - 1)
                ud = plsc.load_gather(u_v, [iota_d + dcol])
                pos_r = plsc.load_gather(rows_v, [iota + neg_per_group, dcol])
                neg_rs = [
                    plsc.load_gather(rows_v, [iota_k + (j - 1), dcol])
                    for j in range(1, n_pairs)
                ]
                return (accs[0] + ud * pos_r,) + tuple(
                    accs[j] + ud * neg_rs[j - 1] for j in range(1, n_pairs)
                )

            zero = jnp.zeros((LANES,), jnp.float32)
            accs = lax.fori_loop(0, D, d_body, tuple(zero for _ in range(n_pairs)))
            sc_v[pl.ds(g * LANES, LANES)] = accs[0]
            for j in range(1, n_pairs):
                sc_v[pl.ds(j * bpw + g * LANES, LANES)] = -accs[j]

        issue(0, rows0, u0, sem0)

        def pair_body(t, carry):
            g0 = 2 * t
            issue(g0 + 1, rows1, u1, sem1)
            drain(rows0, u0, sem0)
            compute(g0, rows0, u0)

            @pl.when(g0 + 2 < G)
            def _():
                issue(g0 + 2, rows0, u0, sem0)

            drain(rows1, u1, sem1)
            compute(g0 + 1, rows1, u1)
            return carry

        lax.fori_loop(0, G // 2, pair_body, 0)
        pltpu.sync_copy(sc_v, out_hbm.at[wid])

    return k(V, u_w, pos_w, neg_w)


def _tc_logsig_sum(scores2d):
    def body(x_ref, o_ref):
        x = x_ref[...]
        ls = jnp.minimum(x, 0.0) - jnp.log(1.0 + jnp.exp(-jnp.abs(x)))
        o_ref[...] = (-jnp.sum(ls)).reshape(1, 1)

    out = pl.pallas_call(
        body,
        out_shape=jax.ShapeDtypeStruct((1, 1), jnp.float32),
    )(scores2d)
    return out[0, 0]


def kernel(pos_u, pos_v, neg_v, V):
    B, D = pos_u.shape
    K = neg_v.shape[1]
    n_pairs = K + 1
    bpw = B // NW
    G = bpw // LANES

    pos_w = pos_v.astype(jnp.int32).reshape(NW, G, LANES)
    neg_w = neg_v.astype(jnp.int32).reshape(NW, -1, NEG_MINOR)
    u_w = pos_u.reshape(NW, bpw * D)
    V128 = jnp.pad(V, ((0, 0), (0, 128 - D)))
    scores = _sc_scores(V128, u_w, pos_w, neg_w, n_pairs, bpw)
    return _tc_logsig_sum(scores.reshape(-1, 512))
